# baseline (speedup 1.0000x reference)
import functools

import jax
import jax.numpy as jnp
from jax.experimental import pallas as pl
from jax.experimental.pallas import tpu as pltpu

ROWS = 16384
D = 2048
BLK = 1024


def _norm_body(x_ref, wt_ref, o_ref):
    xb = x_ref[...]
    wt = wt_ref[...]
    p = jnp.dot(xb, wt, preferred_element_type=jnp.float32)
    o_ref[...] = jnp.sqrt(jnp.sum(p * p, axis=1))[None, None, :]


def kernel(x, routers, expert_idx):
    w = jax.lax.dynamic_index_in_dim(routers, expert_idx, axis=0,
                                     keepdims=False)
    x2 = x.reshape(ROWS, D)
    grid = ROWS // BLK
    out = pl.pallas_call(
        _norm_body,
        grid=(grid,),
        in_specs=[
            pl.BlockSpec((BLK, D), lambda i: (i, 0)),
            pl.BlockSpec((D, 8), lambda i: (0, 0)),
        ],
        out_specs=pl.BlockSpec((1, 1, BLK), lambda i: (i, 0, 0)),
        out_shape=jax.ShapeDtypeStruct((grid, 1, BLK), jnp.float32),
    )(x2, w.T)
    return out.reshape(4, 4096)


# speedup vs baseline: 1.0162x; 1.0162x over previous
import functools

import jax
import jax.numpy as jnp
from jax.experimental import pallas as pl
from jax.experimental.pallas import tpu as pltpu

ROWS = 16384
D = 2048
BLK = 1024


def _norm_body(x_ref, wt_ref, o_ref):
    xb = x_ref[...]
    wt = wt_ref[...]
    p = jnp.dot(xb.astype(jnp.bfloat16), wt.astype(jnp.bfloat16),
                preferred_element_type=jnp.float32)
    o_ref[...] = jnp.sqrt(jnp.sum(p * p, axis=1))[None, None, :]


def kernel(x, routers, expert_idx):
    w = jax.lax.dynamic_index_in_dim(routers, expert_idx, axis=0,
                                     keepdims=False)
    x2 = x.reshape(ROWS, D)
    grid = ROWS // BLK
    out = pl.pallas_call(
        _norm_body,
        grid=(grid,),
        in_specs=[
            pl.BlockSpec((BLK, D), lambda i: (i, 0)),
            pl.BlockSpec((D, 8), lambda i: (0, 0)),
        ],
        out_specs=pl.BlockSpec((1, 1, BLK), lambda i: (i, 0, 0)),
        out_shape=jax.ShapeDtypeStruct((grid, 1, BLK), jnp.float32),
    )(x2, w.T)
    return out.reshape(4, 4096)
